# baseline (speedup 1.0000x reference)
import functools

import jax
import jax.numpy as jnp
from jax import lax
from jax.experimental import pallas as pl
from jax.experimental.pallas import tpu as pltpu
from jax.experimental.pallas import tpu_sc as plsc

_LAMBDA = 0.001
_L = 16
_NC = 2
_NS = 16
_NW = _NC * _NS
_B = 16384
_BPW = _B // _NW
_CHUNK = 128
_NCHUNK = _BPW // _CHUNK
_F = 16
_GROUPS = _BPW // _L
_V = 1000000


def _sc_body(u0_ref, i0_ref, r_ref, eu_ref, ei_ref, ub_ref, ib_ref, avg_ref,
             out_ref,
             uidx_v, iidx_v, utidx_v, itidx_v, r_v, u3_v, i3_v,
             ubias_v, ibias_v, avg_v, res_v, rsem, bsem):
    c = lax.axis_index("c")
    s = lax.axis_index("s")
    wid = s * _NC + c

    pltpu.sync_copy(u0_ref.at[wid], uidx_v)
    pltpu.sync_copy(i0_ref.at[wid], iidx_v)
    pltpu.sync_copy(r_ref.at[wid], r_v)
    pltpu.sync_copy(avg_ref, avg_v)

    copies = []
    for j in range(_NCHUNK):
        sl = pl.ds(j * _CHUNK, _CHUNK)
        copies.append(pltpu.async_copy(ub_ref.at[uidx_v.at[j]],
                                       ubias_v.at[sl], bsem))
        copies.append(pltpu.async_copy(ib_ref.at[iidx_v.at[j]],
                                       ibias_v.at[sl], bsem))

    def translate(g, carry):
        j = g // (_CHUNK // _L)
        o = (g - j * (_CHUNK // _L)) * _L
        osl = pl.ds(o, _L)
        uvec = uidx_v[j, osl]
        ivec = iidx_v[j, osl]
        for f in range(_F):
            utidx_v[f, j, osl] = uvec + f * _V
            itidx_v[f, j, osl] = ivec + f * _V
        return carry

    lax.fori_loop(0, _GROUPS, translate, 0)

    for f in range(_F):
        for j in range(_NCHUNK):
            sl = pl.ds(j * _CHUNK, _CHUNK)
            copies.append(pltpu.async_copy(
                eu_ref.at[utidx_v.at[f, j]], u3_v.at[f, sl], rsem))
            copies.append(pltpu.async_copy(
                ei_ref.at[itidx_v.at[f, j]], i3_v.at[f, sl], rsem))
    for cp in copies:
        cp.wait()

    lane = lax.iota(jnp.int32, _L)
    avgv = avg_v[...]
    zero = jnp.zeros((_L,), jnp.float32)

    def group_body(g, carry):
        sse, u2, i2 = carry
        sl = pl.ds(g * _L, _L)
        sdot = zero
        for f in range(_F):
            cu = u3_v[f, sl]
            ci = i3_v[f, sl]
            sdot = sdot + cu * ci
            u2 = u2 + cu * cu
            i2 = i2 + ci * ci
        e = sdot + ubias_v[sl] + ibias_v[sl] + avgv - r_v[sl]
        sse = sse + e * e
        return sse, u2, i2

    sse, u2, i2 = lax.fori_loop(0, _GROUPS, group_body, (zero, zero, zero))

    sse_s = jnp.sum(sse)
    u2_s = jnp.sum(u2)
    i2_s = jnp.sum(i2)
    res = jnp.where(lane == 0, sse_s,
                    jnp.where(lane == 1, u2_s,
                              jnp.where(lane == 2, i2_s, 0.0)))
    res_v[...] = res
    pltpu.sync_copy(res_v, out_ref.at[wid])


@jax.jit
def kernel(user0, item_i0, ratings, embed_user, embed_item, user_bias_w,
           item_bias_w, avg_rating):
    u0 = user0.reshape(_NW, _NCHUNK, _CHUNK)
    i0 = item_i0.reshape(_NW, _NCHUNK, _CHUNK)
    r = ratings.astype(jnp.float32).reshape(_NW, _BPW)
    eu = embed_user.T.reshape(-1)
    ei = embed_item.T.reshape(-1)
    ub = user_bias_w.reshape(-1)
    ib = item_bias_w.reshape(-1)
    avg16 = jnp.broadcast_to(avg_rating.astype(jnp.float32), (_L,))

    mesh = plsc.VectorSubcoreMesh(core_axis_name="c", subcore_axis_name="s")
    sc_call = pl.kernel(
        _sc_body,
        mesh=mesh,
        compiler_params=pltpu.CompilerParams(
            needs_layout_passes=False, use_tc_tiling_on_sc=False),
        out_type=jax.ShapeDtypeStruct((_NW, _L), jnp.float32),
        scratch_types=[
            pltpu.VMEM((_NCHUNK, _CHUNK), jnp.int32),
            pltpu.VMEM((_NCHUNK, _CHUNK), jnp.int32),
            pltpu.VMEM((_F, _NCHUNK, _CHUNK), jnp.int32),
            pltpu.VMEM((_F, _NCHUNK, _CHUNK), jnp.int32),
            pltpu.VMEM((_BPW,), jnp.float32),
            pltpu.VMEM((_F, _BPW), jnp.float32),
            pltpu.VMEM((_F, _BPW), jnp.float32),
            pltpu.VMEM((_BPW,), jnp.float32),
            pltpu.VMEM((_BPW,), jnp.float32),
            pltpu.VMEM((_L,), jnp.float32),
            pltpu.VMEM((_L,), jnp.float32),
            pltpu.SemaphoreType.DMA,
            pltpu.SemaphoreType.DMA,
        ],
    )
    parts = sc_call(u0, i0, r, eu, ei, ub, ib, avg16)

    sse = parts[:, 0].sum()
    u2 = parts[:, 1].sum()
    i2 = parts[:, 2].sum()
    task_loss = sse / _B
    l2 = _LAMBDA * (u2 / (_B * _F)) + _LAMBDA * (i2 / (_B * _F))
    loss = task_loss + l2
    return (loss, task_loss, l2)


# speedup vs baseline: 18.3373x; 18.3373x over previous
import functools

import jax
import jax.numpy as jnp
from jax import lax
from jax.experimental import pallas as pl
from jax.experimental.pallas import tpu as pltpu
from jax.experimental.pallas import tpu_sc as plsc

_LAMBDA = 0.001
_L = 16
_NC = 2
_NS = 16
_NW = _NC * _NS
_B = 16384
_BPW = _B // _NW
_CHUNK = 128
_NCHUNK = _BPW // _CHUNK
_F = 16
_GPC = _CHUNK // _L
_CL = _L * 128
_V = 1000000


def _sc_body(u0_ref, i0_ref, r_ref, eu_ref, ei_ref, ub_ref, ib_ref, avg_ref,
             dummy_ref, dummyb_ref, out_ref,
             uidx_v, iidx_v, r_v, u3_v, i3_v, ubias_v, ibias_v,
             avg_v, res_v, rsem, bsem):
    c = lax.axis_index("c")
    s = lax.axis_index("s")
    wid = s * _NC + c

    pltpu.sync_copy(u0_ref.at[wid], uidx_v)
    pltpu.sync_copy(i0_ref.at[wid], iidx_v)
    pltpu.sync_copy(r_ref.at[wid], r_v)
    pltpu.sync_copy(avg_ref, avg_v)

    lane = lax.iota(jnp.int32, _L)
    avgv = avg_v[...]
    zero = jnp.zeros((_L,), jnp.float32)

    def load_grp(g):
        j = g // _GPC
        o = (g - j * _GPC) * _L
        return uidx_v[j, pl.ds(o, _L)], iidx_v[j, pl.ds(o, _L)]

    def body(g, carry):
        sse, u2, i2 = carry
        uvec, ivec = load_grp(g)
        for k in range(_L):
            ru = uvec[k]
            ri = ivec[k]
            r0u = pl.multiple_of(
                lax.shift_left(lax.shift_right_logical(ru, 7), 7), 128)
            r0i = pl.multiple_of(
                lax.shift_left(lax.shift_right_logical(ri, 7), 7), 128)
            slot = pl.ds(k * 128, 128)
            pltpu.async_copy(eu_ref.at[:, :, pl.ds(r0u, 128)],
                             u3_v.at[:, :, slot], rsem)
            pltpu.async_copy(ei_ref.at[:, :, pl.ds(r0i, 128)],
                             i3_v.at[:, :, slot], rsem)
            pltpu.async_copy(ub_ref.at[:, pl.ds(r0u, 128)],
                             ubias_v.at[:, slot], bsem)
            pltpu.async_copy(ib_ref.at[:, pl.ds(r0i, 128)],
                             ibias_v.at[:, slot], bsem)
        pltpu.make_async_copy(dummy_ref, u3_v, rsem).wait()
        pltpu.make_async_copy(dummy_ref, i3_v, rsem).wait()
        pltpu.make_async_copy(dummyb_ref, ubias_v, bsem).wait()
        pltpu.make_async_copy(dummyb_ref, ibias_v, bsem).wait()

        lslot = lane * 128
        slu = lslot + lax.bitwise_and(uvec, 127)
        sli = lslot + lax.bitwise_and(ivec, 127)
        zero16 = jnp.zeros((_L,), jnp.int32)
        sdot = zero
        for f in range(_F):
            dt = jnp.full((_L,), f // 8, jnp.int32)
            ds_ = jnp.full((_L,), f % 8, jnp.int32)
            cu = plsc.load_gather(u3_v, [dt, ds_, slu])
            ci = plsc.load_gather(i3_v, [dt, ds_, sli])
            sdot = sdot + cu * ci
            u2 = u2 + cu * cu
            i2 = i2 + ci * ci
        ubv = plsc.load_gather(ubias_v, [zero16, slu])
        ibv = plsc.load_gather(ibias_v, [zero16, sli])
        e = sdot + ubv + ibv + avgv - r_v[pl.ds(g * _L, _L)]
        sse = sse + e * e
        return sse, u2, i2

    sse, u2, i2 = lax.fori_loop(
        0, _BPW // _L, body, (zero, zero, zero))

    sse_s = jnp.sum(sse)
    u2_s = jnp.sum(u2)
    i2_s = jnp.sum(i2)
    res = jnp.where(lane == 0, sse_s,
                    jnp.where(lane == 1, u2_s,
                              jnp.where(lane == 2, i2_s, 0.0)))
    res_v[...] = res
    pltpu.sync_copy(res_v, out_ref.at[wid])


@jax.jit
def kernel(user0, item_i0, ratings, embed_user, embed_item, user_bias_w,
           item_bias_w, avg_rating):
    u0 = user0.reshape(_NW, _NCHUNK, _CHUNK)
    i0 = item_i0.reshape(_NW, _NCHUNK, _CHUNK)
    r = ratings.astype(jnp.float32).reshape(_NW, _BPW)
    eu = embed_user.T.reshape(2, 8, _V)
    ei = embed_item.T.reshape(2, 8, _V)
    ub = user_bias_w.T
    ib = item_bias_w.T
    avg16 = jnp.broadcast_to(avg_rating.astype(jnp.float32), (_L,))
    dummy = jnp.zeros((2, 8, _CL), jnp.float32)
    dummyb = jnp.zeros((1, _CL), jnp.float32)

    mesh = plsc.VectorSubcoreMesh(core_axis_name="c", subcore_axis_name="s")
    sc_call = pl.kernel(
        _sc_body,
        mesh=mesh,
        compiler_params=pltpu.CompilerParams(
            needs_layout_passes=False, use_tc_tiling_on_sc=True),
        out_type=jax.ShapeDtypeStruct((_NW, _L), jnp.float32),
        scratch_types=[
            pltpu.VMEM((_NCHUNK, _CHUNK), jnp.int32),
            pltpu.VMEM((_NCHUNK, _CHUNK), jnp.int32),
            pltpu.VMEM((_BPW,), jnp.float32),
            pltpu.VMEM((2, 8, _CL), jnp.float32),
            pltpu.VMEM((2, 8, _CL), jnp.float32),
            pltpu.VMEM((1, _CL), jnp.float32),
            pltpu.VMEM((1, _CL), jnp.float32),
            pltpu.VMEM((_L,), jnp.float32),
            pltpu.VMEM((_L,), jnp.float32),
            pltpu.SemaphoreType.DMA,
            pltpu.SemaphoreType.DMA,
        ],
    )
    parts = sc_call(u0, i0, r, eu, ei, ub, ib, avg16, dummy, dummyb)

    sse = parts[:, 0].sum()
    u2 = parts[:, 1].sum()
    i2 = parts[:, 2].sum()
    task_loss = sse / _B
    l2 = _LAMBDA * (u2 / (_B * _F)) + _LAMBDA * (i2 / (_B * _F))
    loss = task_loss + l2
    return (loss, task_loss, l2)
